# Initial kernel scaffold; baseline (speedup 1.0000x reference)
import jax
import jax.numpy as jnp
from jax.experimental import pallas as pl


def kernel(x, edge_index, edge_attr, W_node, b_node, W_edge, b_edge, W1s, b1s, bn_s, bn_b, W2s, b2s, W_out, b_out):
    raise NotImplementedError("write your pallas kernel here")


# speedup vs baseline: 3.1812x; 3.1812x over previous
import functools

import jax
import jax.numpy as jnp
from jax import lax
from jax.experimental import pallas as pl
from jax.experimental.pallas import tpu as pltpu
from jax.experimental.pallas import tpu_sc as plsc

NN = 10000
NE = 320000
C = 128
HID = 2 * C
NLAYERS = 4

NC = 2
NS = 16
NW = NC * NS
EPW = NE // NW
CH = 125
NCHUNK = EPW // CH
GRP = 16
NGRP = NCHUNK // GRP
RPT = 624
TAIL = NN - NS * RPT

def _sc_body(h_hbm, e_hbm, src_hbm, dst_hbm, eidx_hbm, zero_hbm, out_hbm,
             src_v, dst_v, eidx_v, hbuf, ebuf, agg_sh, sem):
    cid = lax.axis_index("c")
    sid = lax.axis_index("s")
    wid = cid * NS + sid

    pltpu.sync_copy(zero_hbm.at[pl.ds(sid * RPT, RPT)],
                    agg_sh.at[pl.ds(sid * RPT, RPT)])

    @pl.when(sid == 0)
    def _():
        pltpu.sync_copy(zero_hbm.at[pl.ds(NS * RPT, TAIL)],
                        agg_sh.at[pl.ds(NS * RPT, TAIL)])
    plsc.subcore_barrier()

    def grp_body(g, carry):
        pltpu.sync_copy(src_hbm.at[wid, pl.ds(g * GRP, GRP)], src_v)
        pltpu.sync_copy(dst_hbm.at[wid, pl.ds(g * GRP, GRP)], dst_v)
        pltpu.sync_copy(eidx_hbm.at[wid, pl.ds(g * GRP, GRP)], eidx_v)

        def chunk_body(j, c1):
            cp1 = pltpu.async_copy(h_hbm.at[src_v.at[j]], hbuf, sem)
            cp2 = pltpu.async_copy(e_hbm.at[eidx_v.at[j]], ebuf, sem)
            cp1.wait()
            cp2.wait()

            def row_body(r, c2):
                for k in range(C // 16):
                    sl = pl.ds(k * 16, 16)
                    hbuf[r, sl] = jnp.maximum(hbuf[r, sl] + ebuf[r, sl], 0.0)
                return c2

            lax.fori_loop(0, CH, row_body, 0)
            pltpu.sync_copy(hbuf, agg_sh.at[dst_v.at[j]], add=True)
            return c1

        lax.fori_loop(0, GRP, chunk_body, 0)
        return carry

    lax.fori_loop(0, NGRP, grp_body, 0)
    plsc.subcore_barrier()
    pltpu.sync_copy(agg_sh.at[pl.ds(sid * RPT, RPT)],
                    out_hbm.at[cid, pl.ds(sid * RPT, RPT)])

    @pl.when(sid == 0)
    def _():
        pltpu.sync_copy(agg_sh.at[pl.ds(NS * RPT, TAIL)],
                        out_hbm.at[cid, pl.ds(NS * RPT, TAIL)])


@functools.lru_cache(maxsize=1)
def _sc_msg_agg():
    mesh = plsc.VectorSubcoreMesh(core_axis_name="c", subcore_axis_name="s",
                                  num_cores=NC, num_subcores=NS)
    return pl.kernel(
        _sc_body,
        out_type=jax.ShapeDtypeStruct((NC, NN, C), jnp.float32),
        mesh=mesh,
        scratch_types=[
            pltpu.VMEM((GRP, CH), jnp.int32),
            pltpu.VMEM((GRP, CH), jnp.int32),
            pltpu.VMEM((GRP, CH), jnp.int32),
            pltpu.VMEM((CH, C), jnp.float32),
            pltpu.VMEM((CH, C), jnp.float32),
            pltpu.VMEM_SHARED((NN, C), jnp.float32),
            pltpu.SemaphoreType.DMA,
        ],
    )


def _enc_node_body(x_ref, w_ref, b_ref, o_ref):
    acc = b_ref[...]
    for k in range(3):
        acc = acc + x_ref[:, k:k + 1] * w_ref[k:k + 1, :]
    o_ref[...] = acc


def _enc_edge_body(a_ref, w_ref, b_ref, o_ref):
    acc = b_ref[...]
    for k in range(4):
        acc = acc + a_ref[:, k:k + 1] * w_ref[k:k + 1, :]
    o_ref[...] = acc


def _mlp_body(p_ref, w1_ref, b1_ref, w2_ref, b2_ref, o_ref):
    a = p_ref[0] + p_ref[1]
    z = jnp.dot(a, w1_ref[...], preferred_element_type=jnp.float32)
    z = jnp.maximum(z + b1_ref[...], 0.0)
    o_ref[...] = jnp.dot(z, w2_ref[...],
                         preferred_element_type=jnp.float32) + b2_ref[...]


def _mlp_pool_body(p_ref, w1_ref, b1_ref, w2_ref, b2_ref, wo_ref, bo_ref,
                   o_ref, acc_ref):
    i = pl.program_id(0)
    a = p_ref[0] + p_ref[1]
    z = jnp.dot(a, w1_ref[...], preferred_element_type=jnp.float32)
    z = jnp.maximum(z + b1_ref[...], 0.0)
    h = jnp.dot(z, w2_ref[...],
                preferred_element_type=jnp.float32) + b2_ref[...]
    part = jnp.sum(h, axis=0, keepdims=True)

    @pl.when(i == 0)
    def _():
        acc_ref[...] = part

    @pl.when(i > 0)
    def _():
        acc_ref[...] = acc_ref[...] + part

    @pl.when(i == pl.num_programs(0) - 1)
    def _():
        pooled = acc_ref[...] * (1.0 / NN)
        o_ref[...] = (jnp.sum(pooled * wo_ref[...], axis=1, keepdims=True)
                      + bo_ref[...])


_NB = 1000


def _mlp(parts, w1f, b1f, w2, b2):
    return pl.pallas_call(
        _mlp_body,
        grid=(NN // _NB,),
        in_specs=[
            pl.BlockSpec((NC, _NB, C), lambda i: (0, i, 0)),
            pl.BlockSpec((C, HID), lambda i: (0, 0)),
            pl.BlockSpec((1, HID), lambda i: (0, 0)),
            pl.BlockSpec((HID, C), lambda i: (0, 0)),
            pl.BlockSpec((1, C), lambda i: (0, 0)),
        ],
        out_specs=pl.BlockSpec((_NB, C), lambda i: (i, 0)),
        out_shape=jax.ShapeDtypeStruct((NN, C), jnp.float32),
    )(parts, w1f, b1f, w2, b2)


def _mlp_pool(parts, w1f, b1f, w2, b2, wo, bo):
    return pl.pallas_call(
        _mlp_pool_body,
        grid=(NN // _NB,),
        in_specs=[
            pl.BlockSpec((NC, _NB, C), lambda i: (0, i, 0)),
            pl.BlockSpec((C, HID), lambda i: (0, 0)),
            pl.BlockSpec((1, HID), lambda i: (0, 0)),
            pl.BlockSpec((HID, C), lambda i: (0, 0)),
            pl.BlockSpec((1, C), lambda i: (0, 0)),
            pl.BlockSpec((1, C), lambda i: (0, 0)),
            pl.BlockSpec((1, 1), lambda i: (0, 0)),
        ],
        out_specs=pl.BlockSpec((1, 1), lambda i: (0, 0)),
        out_shape=jax.ShapeDtypeStruct((1, 1), jnp.float32),
        scratch_shapes=[pltpu.VMEM((1, C), jnp.float32)],
    )(parts, w1f, b1f, w2, b2, wo, bo)


def kernel(x, edge_index, edge_attr, W_node, b_node, W_edge, b_edge,
           W1s, b1s, bn_s, bn_b, W2s, b2s, W_out, b_out):
    src1 = edge_index[0].astype(jnp.int32)
    dst1 = edge_index[1].astype(jnp.int32)
    perm = jnp.argsort(dst1).astype(jnp.int32)
    eidx = perm.reshape(CH, NW * NCHUNK).T.reshape(NW, NCHUNK, CH)
    src = src1[eidx]
    dst = dst1[eidx]

    h0 = pl.pallas_call(
        _enc_node_body,
        grid=(5,),
        in_specs=[
            pl.BlockSpec((NN // 5, 3), lambda i: (i, 0)),
            pl.BlockSpec((3, C), lambda i: (0, 0)),
            pl.BlockSpec((1, C), lambda i: (0, 0)),
        ],
        out_specs=pl.BlockSpec((NN // 5, C), lambda i: (i, 0)),
        out_shape=jax.ShapeDtypeStruct((NN, C), jnp.float32),
    )(x, W_node, b_node.reshape(1, C))

    _EB = 4000
    e = pl.pallas_call(
        _enc_edge_body,
        grid=(NE // _EB,),
        in_specs=[
            pl.BlockSpec((_EB, 4), lambda i: (i, 0)),
            pl.BlockSpec((4, C), lambda i: (0, 0)),
            pl.BlockSpec((1, C), lambda i: (0, 0)),
        ],
        out_specs=pl.BlockSpec((_EB, C), lambda i: (i, 0)),
        out_shape=jax.ShapeDtypeStruct((NE, C), jnp.float32),
    )(edge_attr, W_edge, b_edge.reshape(1, C))

    W1f = W1s * bn_s[:, None, :]
    b1f = b1s * bn_s + bn_b
    zero = jnp.zeros((NN, C), jnp.float32)

    h = h0
    for l in range(NLAYERS):
        parts = _sc_msg_agg()(h, e, src, dst, eidx, zero)
        if l < NLAYERS - 1:
            h = _mlp(parts, W1f[l], b1f[l].reshape(1, HID), W2s[l],
                     b2s[l].reshape(1, C))
        else:
            out = _mlp_pool(parts, W1f[l], b1f[l].reshape(1, HID), W2s[l],
                            b2s[l].reshape(1, C), W_out.reshape(1, C),
                            b_out.reshape(1, 1))
    return out
